# Initial kernel scaffold; baseline (speedup 1.0000x reference)
import jax
import jax.numpy as jnp
from jax.experimental import pallas as pl


def kernel(z, attention_out, We0, be0, We1, be1, We2, be2, Wb, bb, Wd0, bd0, Wd1, bd1, Wd2, bd2):
    raise NotImplementedError("write your pallas kernel here")


# speedup vs baseline: 3.2757x; 3.2757x over previous
import jax
import jax.numpy as jnp
from jax.experimental import pallas as pl
from jax.experimental.pallas import tpu as pltpu

_ROWS_PER_BLOCK = 512
_BATCH = 8192


def _quant_body(*refs):
    n = len(refs) // 2
    for w_ref, q_ref in zip(refs[:n], refs[n:]):
        w = w_ref[...]
        gamma = jnp.mean(jnp.abs(w)) + 1e-8
        q_ref[...] = jnp.clip(jnp.round(w / gamma), -1.0, 1.0).astype(
            jnp.bfloat16)


def _quantize_weights(wts):
    return pl.pallas_call(
        _quant_body,
        out_shape=[jax.ShapeDtypeStruct(w.shape, jnp.bfloat16) for w in wts],
        compiler_params=pltpu.CompilerParams(
            vmem_limit_bytes=60 * 1024 * 1024),
    )(*wts)


def _gelu(x):
    return 0.5 * x * (1.0 + jax.lax.erf(x * 0.7071067811865476))


def _mm(x, w):
    return jnp.dot(x.astype(jnp.bfloat16), w,
                   preferred_element_type=jnp.float32)


def _mlp_body(z_ref, a_ref,
              w0_ref, w1_ref, w2_ref, wb_ref, w3_ref, w4_ref, w5_ref,
              b0_ref, b1_ref, b2_ref, bb_ref, b3_ref, b4_ref, b5_ref,
              o_ref):
    w0 = w0_ref[...]
    h = (_mm(z_ref[...], w0[:1024, :])
         + _mm(a_ref[...], w0[1024:, :]))
    h = _gelu(h + b0_ref[...])
    for w_ref, b_ref in ((w1_ref, b1_ref), (w2_ref, b2_ref),
                         (wb_ref, bb_ref), (w3_ref, b3_ref),
                         (w4_ref, b4_ref), (w5_ref, b5_ref)):
        h = _gelu(_mm(h, w_ref[...]) + b_ref[...])
    o_ref[...] = h


@jax.jit
def kernel(z, attention_out, We0, be0, We1, be1, We2, be2, Wb, bb,
           Wd0, bd0, Wd1, bd1, Wd2, bd2):
    weights = [We0.T, We1.T, We2.T, Wb.T, Wd0.T, Wd1.T, Wd2.T]
    qweights = _quantize_weights(weights)
    biases = [b.reshape(1, -1) for b in (be0, be1, be2, bb, bd0, bd1, bd2)]

    r = _ROWS_PER_BLOCK
    grid = (_BATCH // r,)

    def row_block(i):
        return (i, 0)

    def whole(i):
        return (0, 0)

    in_specs = (
        [pl.BlockSpec((r, 1024), row_block),
         pl.BlockSpec((r, 1024), row_block)]
        + [pl.BlockSpec(w.shape, whole) for w in qweights]
        + [pl.BlockSpec(b.shape, whole) for b in biases]
    )
    out = pl.pallas_call(
        _mlp_body,
        grid=grid,
        in_specs=in_specs,
        out_specs=pl.BlockSpec((r, 2048), row_block),
        out_shape=jax.ShapeDtypeStruct((_BATCH, 2048), jnp.float32),
        compiler_params=pltpu.CompilerParams(
            dimension_semantics=("parallel",),
            vmem_limit_bytes=60 * 1024 * 1024),
    )(z, attention_out, *qweights, *biases)
    return out


# speedup vs baseline: 4.1727x; 1.2738x over previous
import jax
import jax.numpy as jnp
from jax.experimental import pallas as pl
from jax.experimental.pallas import tpu as pltpu

_ROWS_PER_BLOCK = 512
_BATCH = 8192


def _quant_body(*refs):
    n = len(refs) // 2
    for w_ref, q_ref in zip(refs[:n], refs[n:]):
        w = w_ref[...]
        gamma = jnp.mean(jnp.abs(w)) + 1e-8
        q_ref[...] = jnp.clip(jnp.round(w / gamma), -1.0, 1.0).astype(
            jnp.bfloat16)


def _quantize_weights(wts):
    return pl.pallas_call(
        _quant_body,
        out_shape=[jax.ShapeDtypeStruct(w.shape, jnp.bfloat16) for w in wts],
        compiler_params=pltpu.CompilerParams(
            vmem_limit_bytes=60 * 1024 * 1024),
    )(*wts)


def _gelu(x):
    return 0.5 * x * (1.0 + jax.lax.erf(x * 0.7071067811865476))


def _mm(x, w):
    return jax.lax.dot_general(
        x.astype(jnp.bfloat16), w, (((1,), (1,)), ((), ())),
        preferred_element_type=jnp.float32)


def _mlp_body(z_ref, a_ref,
              w0_ref, w1_ref, w2_ref, wb_ref, w3_ref, w4_ref, w5_ref,
              b0_ref, b1_ref, b2_ref, bb_ref, b3_ref, b4_ref, b5_ref,
              o_ref):
    w0 = w0_ref[...]
    h = (_mm(z_ref[...], w0[:, :1024])
         + _mm(a_ref[...], w0[:, 1024:]))
    h = _gelu(h + b0_ref[...])
    for w_ref, b_ref in ((w1_ref, b1_ref), (w2_ref, b2_ref),
                         (wb_ref, bb_ref), (w3_ref, b3_ref),
                         (w4_ref, b4_ref), (w5_ref, b5_ref)):
        h = _gelu(_mm(h, w_ref[...]) + b_ref[...])
    o_ref[...] = h


@jax.jit
def kernel(z, attention_out, We0, be0, We1, be1, We2, be2, Wb, bb,
           Wd0, bd0, Wd1, bd1, Wd2, bd2):
    weights = [We0, We1, We2, Wb, Wd0, Wd1, Wd2]
    qweights = _quantize_weights(weights)
    biases = [b.reshape(1, -1) for b in (be0, be1, be2, bb, bd0, bd1, bd2)]

    r = _ROWS_PER_BLOCK
    grid = (_BATCH // r,)

    def row_block(i):
        return (i, 0)

    def whole(i):
        return (0, 0)

    in_specs = (
        [pl.BlockSpec((r, 1024), row_block),
         pl.BlockSpec((r, 1024), row_block)]
        + [pl.BlockSpec(w.shape, whole) for w in qweights]
        + [pl.BlockSpec(b.shape, whole) for b in biases]
    )
    out = pl.pallas_call(
        _mlp_body,
        grid=grid,
        in_specs=in_specs,
        out_specs=pl.BlockSpec((r, 2048), row_block),
        out_shape=jax.ShapeDtypeStruct((_BATCH, 2048), jnp.float32),
        compiler_params=pltpu.CompilerParams(
            dimension_semantics=("parallel",),
            vmem_limit_bytes=60 * 1024 * 1024),
    )(z, attention_out, *qweights, *biases)
    return out


# speedup vs baseline: 4.4162x; 1.0583x over previous
import jax
import jax.numpy as jnp
from jax.experimental import pallas as pl
from jax.experimental.pallas import tpu as pltpu

_ROWS_PER_BLOCK = 1024
_BATCH = 8192


def _quant_body(*refs):
    n = len(refs) // 2
    for w_ref, q_ref in zip(refs[:n], refs[n:]):
        w = w_ref[...]
        gamma = jnp.mean(jnp.abs(w)) + 1e-8
        q_ref[...] = jnp.clip(jnp.round(w / gamma), -1.0, 1.0).astype(
            jnp.bfloat16)


def _quantize_weights(wts):
    return pl.pallas_call(
        _quant_body,
        out_shape=[jax.ShapeDtypeStruct(w.shape, jnp.bfloat16) for w in wts],
        compiler_params=pltpu.CompilerParams(
            vmem_limit_bytes=60 * 1024 * 1024),
    )(*wts)


def _gelu(x):
    return 0.5 * x * (1.0 + jax.lax.erf(x * 0.7071067811865476))


def _mm(x, w):
    return jax.lax.dot_general(
        x.astype(jnp.bfloat16), w, (((1,), (1,)), ((), ())),
        preferred_element_type=jnp.float32)


def _mlp_body(z_ref, a_ref,
              w0_ref, w1_ref, w2_ref, wb_ref, w3_ref, w4_ref, w5_ref,
              b0_ref, b1_ref, b2_ref, bb_ref, b3_ref, b4_ref, b5_ref,
              o_ref):
    w0 = w0_ref[...]
    h = (_mm(z_ref[...], w0[:, :1024])
         + _mm(a_ref[...], w0[:, 1024:]))
    h = _gelu(h + b0_ref[...])
    for w_ref, b_ref in ((w1_ref, b1_ref), (w2_ref, b2_ref),
                         (wb_ref, bb_ref), (w3_ref, b3_ref),
                         (w4_ref, b4_ref), (w5_ref, b5_ref)):
        h = _gelu(_mm(h, w_ref[...]) + b_ref[...])
    o_ref[...] = h


@jax.jit
def kernel(z, attention_out, We0, be0, We1, be1, We2, be2, Wb, bb,
           Wd0, bd0, Wd1, bd1, Wd2, bd2):
    weights = [We0, We1, We2, Wb, Wd0, Wd1, Wd2]
    qweights = _quantize_weights(weights)
    biases = [b.reshape(1, -1) for b in (be0, be1, be2, bb, bd0, bd1, bd2)]

    r = _ROWS_PER_BLOCK
    grid = (_BATCH // r,)

    def row_block(i):
        return (i, 0)

    def whole(i):
        return (0, 0)

    in_specs = (
        [pl.BlockSpec((r, 1024), row_block),
         pl.BlockSpec((r, 1024), row_block)]
        + [pl.BlockSpec(w.shape, whole) for w in qweights]
        + [pl.BlockSpec(b.shape, whole) for b in biases]
    )
    out = pl.pallas_call(
        _mlp_body,
        grid=grid,
        in_specs=in_specs,
        out_specs=pl.BlockSpec((r, 2048), row_block),
        out_shape=jax.ShapeDtypeStruct((_BATCH, 2048), jnp.float32),
        compiler_params=pltpu.CompilerParams(
            dimension_semantics=("parallel",),
            vmem_limit_bytes=60 * 1024 * 1024),
    )(z, attention_out, *qweights, *biases)
    return out
